# baseline (speedup 1.0000x reference)
import functools
import math

import jax
import jax.numpy as jnp
from jax import lax
from jax.experimental import pallas as pl
from jax.experimental.pallas import tpu as pltpu


def _router_body(nblk, cap, n_tokens, x_ref, wr_ref, br_ref,
                 dest_ref, wn_ref, carry_ref):
    b = pl.program_id(0)
    tb = x_ref.shape[0]
    e = wr_ref.shape[1]

    @pl.when(b == 0)
    def _():
        carry_ref[...] = jnp.zeros_like(carry_ref)

    logits = jnp.dot(x_ref[...], wr_ref[...],
                     preferred_element_type=jnp.float32) + br_ref[...]
    m = jnp.max(logits, axis=1, keepdims=True)
    s = jnp.sum(jnp.exp(logits - m), axis=1, keepdims=True)
    p = 1.0 / s
    t = p / (p + 1e-9)
    wn = t / (t + 1e-9)

    col = lax.broadcasted_iota(jnp.int32, (tb, e), 1)
    e_idx = jnp.min(jnp.where(logits == m, col, e), axis=1, keepdims=True)

    onehot = (col == e_idx).astype(jnp.float32)
    ii = lax.broadcasted_iota(jnp.int32, (tb, tb), 0)
    jj = lax.broadcasted_iota(jnp.int32, (tb, tb), 1)
    lstrict = (jj < ii).astype(jnp.float32)
    csum = jnp.dot(lstrict, onehot, preferred_element_type=jnp.float32)
    pos = jnp.sum(onehot * (csum + carry_ref[...]), axis=1, keepdims=True)
    carry_ref[...] += jnp.sum(onehot, axis=0, keepdims=True)

    pos_i = pos.astype(jnp.int32)
    keep = pos_i < cap
    dest_ref[...] = jnp.where(keep, e_idx * cap + pos_i, n_tokens)
    wn_ref[...] = jnp.where(keep, wn, 0.0)


def _router(x2, wr, br, cap, nblk):
    n, d = x2.shape
    e = wr.shape[1]
    tb = n // nblk
    return pl.pallas_call(
        functools.partial(_router_body, nblk, cap, n),
        grid=(nblk,),
        in_specs=[
            pl.BlockSpec((tb, d), lambda b: (b, 0)),
            pl.BlockSpec((d, e), lambda b: (0, 0)),
            pl.BlockSpec((1, e), lambda b: (0, 0)),
        ],
        out_specs=[
            pl.BlockSpec((tb, 1), lambda b: (b, 0)),
            pl.BlockSpec((tb, 1), lambda b: (b, 0)),
        ],
        out_shape=[
            jax.ShapeDtypeStruct((n, 1), jnp.int32),
            jax.ShapeDtypeStruct((n, 1), jnp.float32),
        ],
        scratch_shapes=[pltpu.VMEM((1, e), jnp.float32)],
    )(x2, wr, br.reshape(1, e))


def _moe_body(ne, nf, cap, x_ref, dest_ref, wn_ref, w1_ref, b1_ref,
              w2_ref, b2_ref, y_ref, yacc_ref, ein_ref, eacc_ref):
    e = pl.program_id(0)
    f = pl.program_id(1)
    n = x_ref.shape[0]

    slot = lax.broadcasted_iota(jnp.int32, (n, cap), 1) + e * cap
    pt = (dest_ref[...] == slot).astype(jnp.float32)

    @pl.when(f == 0)
    def _():
        ein_ref[...] = lax.dot_general(
            pt, x_ref[...], (((0,), (0,)), ((), ())),
            preferred_element_type=jnp.float32)

    h = jnp.dot(ein_ref[...], w1_ref[0],
                preferred_element_type=jnp.float32) + b1_ref[0]
    g = 0.5 * h * (1.0 + lax.erf(h * (1.0 / math.sqrt(2.0))))
    part = jnp.dot(g, w2_ref[0], preferred_element_type=jnp.float32)

    @pl.when(f == 0)
    def _():
        eacc_ref[...] = jnp.zeros_like(eacc_ref)

    eacc_ref[...] += part

    @pl.when(f == nf - 1)
    def _():
        eo = eacc_ref[...] + b2_ref[0]
        contrib = jnp.dot(pt * wn_ref[...], eo,
                          preferred_element_type=jnp.float32)

        @pl.when(e == 0)
        def _():
            yacc_ref[...] = jnp.zeros_like(yacc_ref)

        yacc_ref[...] += contrib

        @pl.when(e == ne - 1)
        def _():
            y_ref[...] = yacc_ref[...]


def _moe(x2, dest, wn, w1, b1, w2, b2, cap, tf):
    n, d = x2.shape
    e, _, dff = w1.shape
    nf = dff // tf
    return pl.pallas_call(
        functools.partial(_moe_body, e, nf, cap),
        grid=(e, nf),
        in_specs=[
            pl.BlockSpec((n, d), lambda i, f: (0, 0)),
            pl.BlockSpec((n, 1), lambda i, f: (0, 0)),
            pl.BlockSpec((n, 1), lambda i, f: (0, 0)),
            pl.BlockSpec((1, d, tf), lambda i, f: (i, 0, f)),
            pl.BlockSpec((1, 1, tf), lambda i, f: (i, 0, f)),
            pl.BlockSpec((1, tf, d), lambda i, f: (i, f, 0)),
            pl.BlockSpec((1, 1, d), lambda i, f: (i, 0, 0)),
        ],
        out_specs=pl.BlockSpec((n, d), lambda i, f: (0, 0)),
        out_shape=jax.ShapeDtypeStruct((n, d), jnp.float32),
        scratch_shapes=[
            pltpu.VMEM((n, d), jnp.float32),
            pltpu.VMEM((cap, d), jnp.float32),
            pltpu.VMEM((cap, d), jnp.float32),
        ],
        compiler_params=pltpu.CompilerParams(
            dimension_semantics=("arbitrary", "arbitrary")),
    )(x2, dest, wn, w1, b1.reshape(e, 1, dff), w2, b2.reshape(e, 1, d))


def kernel(x, Wr, br, W1, b1, W2, b2):
    orig_shape = x.shape
    d = orig_shape[-1]
    x2 = x.reshape(-1, d)
    n = x2.shape[0]
    e = Wr.shape[1]
    cap = max(1, int(math.ceil(float(n) / float(e))))

    dest, wn = _router(x2, Wr, br, cap, nblk=8)
    y = _moe(x2, dest, wn, W1, b1, W2, b2, cap, tf=1536)
    return y.reshape(orig_shape)


# speedup vs baseline: 1.1886x; 1.1886x over previous
import functools
import math

import jax
import jax.numpy as jnp
from jax import lax
from jax.experimental import pallas as pl
from jax.experimental.pallas import tpu as pltpu


def _router_body(nblk, cap, n_tokens, x_ref, wr_ref, br_ref,
                 dest_ref, wn_ref, carry_ref):
    b = pl.program_id(0)
    tb = x_ref.shape[0]
    e = wr_ref.shape[1]

    @pl.when(b == 0)
    def _():
        carry_ref[...] = jnp.zeros_like(carry_ref)

    logits = jnp.dot(x_ref[...], wr_ref[...],
                     preferred_element_type=jnp.float32) + br_ref[...]
    m = jnp.max(logits, axis=1, keepdims=True)
    s = jnp.sum(jnp.exp(logits - m), axis=1, keepdims=True)
    p = 1.0 / s
    t = p / (p + 1e-9)
    wn = t / (t + 1e-9)

    col = lax.broadcasted_iota(jnp.int32, (tb, e), 1)
    e_idx = jnp.min(jnp.where(logits == m, col, e), axis=1, keepdims=True)

    onehot = (col == e_idx).astype(jnp.float32)
    ii = lax.broadcasted_iota(jnp.int32, (tb, tb), 0)
    jj = lax.broadcasted_iota(jnp.int32, (tb, tb), 1)
    lstrict = (jj < ii).astype(jnp.float32)
    csum = jnp.dot(lstrict, onehot, preferred_element_type=jnp.float32)
    pos = jnp.sum(onehot * (csum + carry_ref[...]), axis=1, keepdims=True)
    carry_ref[...] += jnp.sum(onehot, axis=0, keepdims=True)

    pos_i = pos.astype(jnp.int32)
    keep = pos_i < cap
    dest_ref[...] = jnp.where(keep, e_idx * cap + pos_i, n_tokens)
    wn_ref[...] = jnp.where(keep, wn, 0.0)


def _router(x2, wr, br, cap, nblk):
    n, d = x2.shape
    e = wr.shape[1]
    tb = n // nblk
    return pl.pallas_call(
        functools.partial(_router_body, nblk, cap, n),
        grid=(nblk,),
        in_specs=[
            pl.BlockSpec((tb, d), lambda b: (b, 0)),
            pl.BlockSpec((d, e), lambda b: (0, 0)),
            pl.BlockSpec((1, e), lambda b: (0, 0)),
        ],
        out_specs=[
            pl.BlockSpec((tb, 1), lambda b: (b, 0)),
            pl.BlockSpec((tb, 1), lambda b: (b, 0)),
        ],
        out_shape=[
            jax.ShapeDtypeStruct((n, 1), jnp.int32),
            jax.ShapeDtypeStruct((n, 1), jnp.float32),
        ],
        scratch_shapes=[pltpu.VMEM((1, e), jnp.float32)],
    )(x2, wr, br.reshape(1, e))


def _moe_body(cap, x_ref, dest_ref, w1_ref, b1_ref, w2_ref, b2_ref,
              eo_ref):
    e = pl.program_id(0)
    n = x_ref.shape[0]

    slot = lax.broadcasted_iota(jnp.int32, (n, cap), 1) + e * cap
    pt = (dest_ref[...] == slot).astype(jnp.float32)
    ein = lax.dot_general(
        pt, x_ref[...], (((0,), (0,)), ((), ())),
        preferred_element_type=jnp.float32)

    h = jnp.dot(ein, w1_ref[0],
                preferred_element_type=jnp.float32) + b1_ref[0]
    g = 0.5 * h * (1.0 + lax.erf(h * (1.0 / math.sqrt(2.0))))
    eo_ref[0] = jnp.dot(g, w2_ref[0],
                        preferred_element_type=jnp.float32) + b2_ref[0]


def _moe(x2, dest, w1, b1, w2, b2, cap):
    n, d = x2.shape
    e, _, dff = w1.shape
    return pl.pallas_call(
        functools.partial(_moe_body, cap),
        grid=(e,),
        in_specs=[
            pl.BlockSpec((n, d), lambda i: (0, 0)),
            pl.BlockSpec((n, 1), lambda i: (0, 0)),
            pl.BlockSpec((1, d, dff), lambda i: (i, 0, 0)),
            pl.BlockSpec((1, 1, dff), lambda i: (i, 0, 0)),
            pl.BlockSpec((1, dff, d), lambda i: (i, 0, 0)),
            pl.BlockSpec((1, 1, d), lambda i: (i, 0, 0)),
        ],
        out_specs=pl.BlockSpec((1, cap, d), lambda i: (i, 0, 0)),
        out_shape=jax.ShapeDtypeStruct((e, cap, d), jnp.float32),
        compiler_params=pltpu.CompilerParams(
            dimension_semantics=("arbitrary",)),
    )(x2, dest, w1, b1.reshape(e, 1, dff), w2, b2.reshape(e, 1, d))


def _combine_body(dest_ref, wn_ref, eo_ref, y_ref):
    ns = eo_ref.shape[0]
    tb = dest_ref.shape[0]
    slot = lax.broadcasted_iota(jnp.int32, (tb, ns), 1)
    cm = (dest_ref[...] == slot).astype(jnp.float32) * wn_ref[...]
    y_ref[...] = jnp.dot(cm, eo_ref[...], preferred_element_type=jnp.float32)


def _combine(dest, wn, eo2, nblk):
    ns, d = eo2.shape
    n = dest.shape[0]
    tb = n // nblk
    return pl.pallas_call(
        _combine_body,
        grid=(nblk,),
        in_specs=[
            pl.BlockSpec((tb, 1), lambda b: (b, 0)),
            pl.BlockSpec((tb, 1), lambda b: (b, 0)),
            pl.BlockSpec((ns, d), lambda b: (0, 0)),
        ],
        out_specs=pl.BlockSpec((tb, d), lambda b: (b, 0)),
        out_shape=jax.ShapeDtypeStruct((n, d), jnp.float32),
    )(dest, wn, eo2)


def kernel(x, Wr, br, W1, b1, W2, b2):
    orig_shape = x.shape
    d = orig_shape[-1]
    x2 = x.reshape(-1, d)
    n = x2.shape[0]
    e = Wr.shape[1]
    cap = max(1, int(math.ceil(float(n) / float(e))))

    dest, wn = _router(x2, Wr, br, cap, nblk=8)
    eo = _moe(x2, dest, W1, b1, W2, b2, cap)
    y = _combine(dest, wn, eo.reshape(e * cap, d), nblk=8)
    return y.reshape(orig_shape)
